# baseline (speedup 1.0000x reference)
import functools

import jax
import jax.numpy as jnp
from jax import lax
from jax.experimental import pallas as pl
from jax.experimental.pallas import tpu as pltpu
from jax.experimental.pallas import tpu_sc as plsc

_K = 256
_N = 32768
_B = 128
_NC = 2
_NS = 16
_NW = _NC * _NS
_ROWS_PER_W = _B // _NW
_LANES = 16
_CHUNKS = _N // _LANES
_UNROLL = 8


def _level_histogram(row_v, hist_v, p, shift, mask_shift, bmask):
    lane_base = lax.iota(jnp.int32, _LANES) * 256
    ones = jnp.ones((_LANES,), jnp.int32)
    p_vec = jnp.full((_LANES,), p, jnp.int32)
    hi_mask = jnp.int32(0x7FFFFFFF)

    def zbody(i, c):
        hist_v[pl.ds(i * _LANES, _LANES)] = jnp.zeros((_LANES,), jnp.int32)
        return c

    lax.fori_loop(0, 256, zbody, 0)

    def hbody(i, c):
        base = i * (_LANES * _UNROLL)
        for cc in range(_UNROLL):
            xv = row_v[pl.ds(base + cc * _LANES, _LANES)]
            u = lax.bitcast_convert_type(xv, jnp.int32) & hi_mask
            msk = (u >> mask_shift) == p_vec
            b = (u >> shift) & bmask
            vals = jnp.where(msk, ones, jnp.zeros((_LANES,), jnp.int32))
            plsc.addupdate_scatter(hist_v, [lane_base + b], vals)
        return c

    lax.fori_loop(0, _CHUNKS // _UNROLL, hbody, 0)


def _level_scan(hist_v, kk):
    def mbody(lane, accs):
        base = lane * 256
        return tuple(
            acc + hist_v[pl.ds(base + j * _LANES, _LANES)]
            for j, acc in enumerate(accs)
        )

    merged = lax.fori_loop(
        0, _LANES, mbody,
        tuple(jnp.zeros((_LANES,), jnp.int32) for _ in range(16)),
    )

    zero = jnp.int32(0)
    s_after = zero
    sel_j = zero
    sel_after = zero
    sel_vec = jnp.zeros((_LANES,), jnp.int32)
    for j in reversed(range(16)):
        vsj = jnp.sum(merged[j])
        crossing = jnp.logical_and(s_after < kk, s_after + vsj >= kk)
        sel_j = jnp.where(crossing, jnp.int32(j), sel_j)
        sel_after = jnp.where(crossing, s_after, sel_after)
        sel_vec = jnp.where(crossing, merged[j], sel_vec)
        s_after = s_after + vsj

    suffix = jnp.flip(plsc.cumsum(jnp.flip(sel_vec, 0)), 0)
    kk_vec = jnp.full((_LANES,), kk, jnp.int32)
    sel_after_vec = jnp.full((_LANES,), sel_after, jnp.int32)
    cond = (sel_after_vec + suffix) >= kk_vec
    pc = plsc.all_reduce_population_count(cond)
    lane_vec = pc - 1
    lanes = lax.iota(jnp.int32, _LANES)
    above_in_vec = jnp.sum(
        jnp.where(lanes > lane_vec, sel_vec, jnp.zeros((_LANES,), jnp.int32))
    )
    lane_s = jnp.max(lane_vec)
    b = sel_j * 16 + lane_s
    count_above = sel_after + above_in_vec
    h_at = jnp.sum(
        jnp.where(lanes == lane_vec, sel_vec, jnp.zeros((_LANES,), jnp.int32))
    )
    return b, count_above, h_at


def _select_and_mask_row(row_v, hist_v):
    kk = jnp.int32(_K)
    p = jnp.int32(0)
    for shift in (23, 15, 7):
        _level_histogram(row_v, hist_v, p, shift, shift + 8, jnp.int32(0xFF))
        b, count_above, _ = _level_scan(hist_v, kk)
        kk = kk - count_above
        p = (p << 8) | b
    _level_histogram(row_v, hist_v, p, 0, 7, jnp.int32(0x7F))
    b, count_above, h_at = _level_scan(hist_v, kk)
    t = (p << 7) | b
    e = h_at - (kk - count_above)

    tv = jnp.full((_LANES,), t, jnp.int32)
    hi_mask = jnp.int32(0x7FFFFFFF)
    zf = jnp.zeros((_LANES,), jnp.float32)

    def obody(i, c):
        base = i * (_LANES * _UNROLL)
        for cc in range(_UNROLL):
            sl = pl.ds(base + cc * _LANES, _LANES)
            xv = row_v[sl]
            u = lax.bitcast_convert_type(xv, jnp.int32) & hi_mask
            row_v[sl] = jnp.where(u >= tv, xv, zf)
        return c

    lax.fori_loop(0, _CHUNKS // _UNROLL, obody, 0)

    @pl.when(e > 0)
    def _fixup():
        e_vec = jnp.full((_LANES,), e, jnp.int32)

        def fbody(i, cnt):
            sl = pl.ds(i * _LANES, _LANES)
            xv = row_v[sl]
            u = lax.bitcast_convert_type(xv, jnp.int32) & hi_mask
            eq = u == tv
            inc = plsc.cumsum(eq.astype(jnp.int32))
            occ = cnt + inc
            row_v[sl] = jnp.where(
                jnp.logical_and(eq, occ <= e_vec), zf, xv
            )
            return cnt + jnp.max(inc)

        lax.fori_loop(0, _CHUNKS, fbody, jnp.int32(0))


def _sc_body(x_hbm, out_hbm, row_v, hist_v):
    wid = lax.axis_index("c") * _NS + lax.axis_index("s")

    def rbody(rr, c):
        row = wid * _ROWS_PER_W + rr
        pltpu.sync_copy(x_hbm.at[row], row_v)
        _select_and_mask_row(row_v, hist_v)
        pltpu.sync_copy(row_v, out_hbm.at[row])
        return c

    lax.fori_loop(0, _ROWS_PER_W, rbody, 0)


def kernel(x):
    mesh = plsc.VectorSubcoreMesh(core_axis_name="c", subcore_axis_name="s")
    f = pl.kernel(
        _sc_body,
        out_type=jax.ShapeDtypeStruct((_B, _N), jnp.float32),
        mesh=mesh,
        compiler_params=pltpu.CompilerParams(needs_layout_passes=False),
        scratch_types=[
            pltpu.VMEM((_N,), jnp.float32),
            pltpu.VMEM((_LANES * 256,), jnp.int32),
        ],
    )
    return f(x)


# speedup vs baseline: 2.6617x; 2.6617x over previous
import functools

import jax
import jax.numpy as jnp
from jax import lax
from jax.experimental import pallas as pl
from jax.experimental.pallas import tpu as pltpu
from jax.experimental.pallas import tpu_sc as plsc

_K = 256
_N = 32768
_B = 128
_NC = 2
_NS = 16
_NW = _NC * _NS
_ROWS_PER_W = _B // _NW
_LANES = 16
_CHUNKS = _N // _LANES
_UNROLL = 8


def _level_histogram(row_v, hist_v, p, shift, mask_shift, bmask):
    lane_base = lax.iota(jnp.int32, _LANES) * 256
    ones = jnp.ones((_LANES,), jnp.int32)
    p_vec = jnp.full((_LANES,), p, jnp.int32)
    hi_mask = jnp.int32(0x7FFFFFFF)

    @plsc.parallel_loop(0, 256, unroll=_UNROLL)
    def _zero(i):
        hist_v[pl.ds(i * _LANES, _LANES)] = jnp.zeros((_LANES,), jnp.int32)

    @plsc.parallel_loop(0, _CHUNKS, unroll=_UNROLL)
    def _hist(i):
        xv = row_v[pl.ds(i * _LANES, _LANES)]
        u = lax.bitcast_convert_type(xv, jnp.int32) & hi_mask
        msk = (u >> mask_shift) == p_vec
        b = (u >> shift) & bmask
        vals = jnp.where(msk, ones, jnp.zeros((_LANES,), jnp.int32))
        plsc.addupdate_scatter(hist_v, [lane_base + b], vals)


def _level_scan(hist_v, kk):
    def mbody(lane, accs):
        base = lane * 256
        return tuple(
            acc + hist_v[pl.ds(base + j * _LANES, _LANES)]
            for j, acc in enumerate(accs)
        )

    merged = lax.fori_loop(
        0, _LANES, mbody,
        tuple(jnp.zeros((_LANES,), jnp.int32) for _ in range(16)),
    )

    zero = jnp.int32(0)
    s_after = zero
    sel_j = zero
    sel_after = zero
    sel_vec = jnp.zeros((_LANES,), jnp.int32)
    for j in reversed(range(16)):
        vsj = jnp.sum(merged[j])
        crossing = jnp.logical_and(s_after < kk, s_after + vsj >= kk)
        sel_j = jnp.where(crossing, jnp.int32(j), sel_j)
        sel_after = jnp.where(crossing, s_after, sel_after)
        sel_vec = jnp.where(crossing, merged[j], sel_vec)
        s_after = s_after + vsj

    suffix = jnp.flip(plsc.cumsum(jnp.flip(sel_vec, 0)), 0)
    kk_vec = jnp.full((_LANES,), kk, jnp.int32)
    sel_after_vec = jnp.full((_LANES,), sel_after, jnp.int32)
    cond = (sel_after_vec + suffix) >= kk_vec
    pc = plsc.all_reduce_population_count(cond)
    lane_vec = pc - 1
    lanes = lax.iota(jnp.int32, _LANES)
    above_in_vec = jnp.sum(
        jnp.where(lanes > lane_vec, sel_vec, jnp.zeros((_LANES,), jnp.int32))
    )
    lane_s = jnp.max(lane_vec)
    b = sel_j * 16 + lane_s
    count_above = sel_after + above_in_vec
    h_at = jnp.sum(
        jnp.where(lanes == lane_vec, sel_vec, jnp.zeros((_LANES,), jnp.int32))
    )
    return b, count_above, h_at


def _select_and_mask_row(row_v, hist_v):
    kk = jnp.int32(_K)
    p = jnp.int32(0)
    for shift in (23, 15, 7):
        _level_histogram(row_v, hist_v, p, shift, shift + 8, jnp.int32(0xFF))
        b, count_above, _ = _level_scan(hist_v, kk)
        kk = kk - count_above
        p = (p << 8) | b
    _level_histogram(row_v, hist_v, p, 0, 7, jnp.int32(0x7F))
    b, count_above, h_at = _level_scan(hist_v, kk)
    t = (p << 7) | b
    e = h_at - (kk - count_above)

    tv = jnp.full((_LANES,), t, jnp.int32)
    hi_mask = jnp.int32(0x7FFFFFFF)
    zf = jnp.zeros((_LANES,), jnp.float32)

    @plsc.parallel_loop(0, _CHUNKS, unroll=_UNROLL)
    def _out(i):
        sl = pl.ds(i * _LANES, _LANES)
        xv = row_v[sl]
        u = lax.bitcast_convert_type(xv, jnp.int32) & hi_mask
        row_v[sl] = jnp.where(u >= tv, xv, zf)

    @pl.when(e > 0)
    def _fixup():
        e_vec = jnp.full((_LANES,), e, jnp.int32)

        def fbody(i, cnt):
            sl = pl.ds(i * _LANES, _LANES)
            xv = row_v[sl]
            u = lax.bitcast_convert_type(xv, jnp.int32) & hi_mask
            eq = u == tv
            inc = plsc.cumsum(eq.astype(jnp.int32))
            occ = cnt + inc
            row_v[sl] = jnp.where(
                jnp.logical_and(eq, occ <= e_vec), zf, xv
            )
            return cnt + jnp.max(inc)

        lax.fori_loop(0, _CHUNKS, fbody, jnp.int32(0))


def _sc_body(x_hbm, out_hbm, row_v, hist_v):
    wid = lax.axis_index("c") * _NS + lax.axis_index("s")

    def rbody(rr, c):
        row = wid * _ROWS_PER_W + rr
        pltpu.sync_copy(x_hbm.at[row], row_v)
        _select_and_mask_row(row_v, hist_v)
        pltpu.sync_copy(row_v, out_hbm.at[row])
        return c

    lax.fori_loop(0, _ROWS_PER_W, rbody, 0)


def kernel(x):
    mesh = plsc.VectorSubcoreMesh(core_axis_name="c", subcore_axis_name="s")
    f = pl.kernel(
        _sc_body,
        out_type=jax.ShapeDtypeStruct((_B, _N), jnp.float32),
        mesh=mesh,
        compiler_params=pltpu.CompilerParams(needs_layout_passes=False),
        scratch_types=[
            pltpu.VMEM((_N,), jnp.float32),
            pltpu.VMEM((_LANES * 256,), jnp.int32),
        ],
    )
    return f(x)


# speedup vs baseline: 3.0992x; 1.1643x over previous
import functools

import jax
import jax.numpy as jnp
from jax import lax
from jax.experimental import pallas as pl
from jax.experimental.pallas import tpu as pltpu
from jax.experimental.pallas import tpu_sc as plsc

_K = 256
_N = 32768
_B = 128
_NC = 2
_NS = 16
_NW = _NC * _NS
_ROWS_PER_W = _B // _NW
_LANES = 16
_CHUNKS = _N // _LANES
_UNROLL = 8


def _level_histogram(row_v, hist_v, p, shift, mask_shift, bmask):
    lane_ids = lax.iota(jnp.int32, _LANES)
    ones = jnp.ones((_LANES,), jnp.int32)
    p_vec = jnp.full((_LANES,), p, jnp.int32)
    hi_mask = jnp.int32(0x7FFFFFFF)

    @plsc.parallel_loop(0, 256, unroll=_UNROLL)
    def _zero(i):
        hist_v[pl.ds(i * _LANES, _LANES)] = jnp.zeros((_LANES,), jnp.int32)

    @plsc.parallel_loop(0, _CHUNKS, unroll=_UNROLL)
    def _hist(i):
        xv = row_v[pl.ds(i * _LANES, _LANES)]
        u = lax.bitcast_convert_type(xv, jnp.int32) & hi_mask
        msk = (u >> mask_shift) == p_vec
        b = (u >> shift) & bmask
        vals = jnp.where(msk, ones, jnp.zeros((_LANES,), jnp.int32))
        plsc.addupdate_scatter(hist_v, [b * _LANES + lane_ids], vals)


def _level_scan(hist_v, kk):
    def mbody(i, accs):
        return tuple(
            acc + hist_v[pl.ds((j * 16 + i) * _LANES, _LANES)]
            for j, acc in enumerate(accs)
        )

    g = lax.fori_loop(
        0, 16, mbody,
        tuple(jnp.zeros((_LANES,), jnp.int32) for _ in range(16)),
    )

    gt = [jnp.sum(gj) for gj in g]

    zero = jnp.int32(0)
    s_after = zero
    sel_j = zero
    sel_after = zero
    for j in reversed(range(16)):
        crossing = jnp.logical_and(s_after < kk, s_after + gt[j] >= kk)
        sel_j = jnp.where(crossing, jnp.int32(j), sel_j)
        sel_after = jnp.where(crossing, s_after, sel_after)
        s_after = s_after + gt[j]

    base = sel_j * (16 * _LANES)
    tb = [jnp.sum(hist_v[pl.ds(base + i * _LANES, _LANES)]) for i in range(16)]
    s_after2 = sel_after
    sel_i = zero
    count_above = zero
    h_at = zero
    for i in reversed(range(16)):
        crossing = jnp.logical_and(s_after2 < kk, s_after2 + tb[i] >= kk)
        sel_i = jnp.where(crossing, jnp.int32(i), sel_i)
        count_above = jnp.where(crossing, s_after2, count_above)
        h_at = jnp.where(crossing, tb[i], h_at)
        s_after2 = s_after2 + tb[i]

    b = sel_j * 16 + sel_i
    return b, count_above, h_at


def _select_and_mask_row(row_v, hist_v):
    kk = jnp.int32(_K)
    p = jnp.int32(0)
    for shift in (23, 15, 7):
        _level_histogram(row_v, hist_v, p, shift, shift + 8, jnp.int32(0xFF))
        b, count_above, _ = _level_scan(hist_v, kk)
        kk = kk - count_above
        p = (p << 8) | b
    _level_histogram(row_v, hist_v, p, 0, 7, jnp.int32(0x7F))
    b, count_above, h_at = _level_scan(hist_v, kk)
    t = (p << 7) | b
    e = h_at - (kk - count_above)

    tv = jnp.full((_LANES,), t, jnp.int32)
    hi_mask = jnp.int32(0x7FFFFFFF)
    zf = jnp.zeros((_LANES,), jnp.float32)

    @plsc.parallel_loop(0, _CHUNKS, unroll=_UNROLL)
    def _out(i):
        sl = pl.ds(i * _LANES, _LANES)
        xv = row_v[sl]
        u = lax.bitcast_convert_type(xv, jnp.int32) & hi_mask
        row_v[sl] = jnp.where(u >= tv, xv, zf)

    @pl.when(e > 0)
    def _fixup():
        e_vec = jnp.full((_LANES,), e, jnp.int32)

        def fbody(i, cnt):
            sl = pl.ds(i * _LANES, _LANES)
            xv = row_v[sl]
            u = lax.bitcast_convert_type(xv, jnp.int32) & hi_mask
            eq = u == tv
            inc = plsc.cumsum(eq.astype(jnp.int32))
            occ = cnt + inc
            row_v[sl] = jnp.where(
                jnp.logical_and(eq, occ <= e_vec), zf, xv
            )
            return cnt + jnp.max(inc)

        lax.fori_loop(0, _CHUNKS, fbody, jnp.int32(0))


def _sc_body(x_hbm, out_hbm, row_v, hist_v):
    wid = lax.axis_index("c") * _NS + lax.axis_index("s")

    def rbody(rr, c):
        row = wid * _ROWS_PER_W + rr
        pltpu.sync_copy(x_hbm.at[row], row_v)
        _select_and_mask_row(row_v, hist_v)
        pltpu.sync_copy(row_v, out_hbm.at[row])
        return c

    lax.fori_loop(0, _ROWS_PER_W, rbody, 0)


def kernel(x):
    mesh = plsc.VectorSubcoreMesh(core_axis_name="c", subcore_axis_name="s")
    f = pl.kernel(
        _sc_body,
        out_type=jax.ShapeDtypeStruct((_B, _N), jnp.float32),
        mesh=mesh,
        compiler_params=pltpu.CompilerParams(needs_layout_passes=False),
        scratch_types=[
            pltpu.VMEM((_N,), jnp.float32),
            pltpu.VMEM((_LANES * 256,), jnp.int32),
        ],
    )
    return f(x)


# speedup vs baseline: 3.3218x; 1.0718x over previous
import jax
import jax.numpy as jnp
from jax import lax
from jax.experimental import pallas as pl
from jax.experimental.pallas import tpu as pltpu
from jax.experimental.pallas import tpu_sc as plsc

_K = 256
_N = 32768
_B = 128
_NC = 2
_NS = 16
_NW = _NC * _NS
_ROWS_PER_W = _B // _NW
_LANES = 16
_CHUNKS = _N // _LANES
_UNROLL = 8
_HI = 0x7FFFFFFF
_DUMP = _N


def _zero_hist(hist_v):
    @plsc.parallel_loop(0, 256, unroll=_UNROLL)
    def _zero(i):
        hist_v[pl.ds(i * _LANES, _LANES)] = jnp.zeros((_LANES,), jnp.int32)


def _level_scan(hist_v, kk):
    def mbody(i, accs):
        return tuple(
            acc + hist_v[pl.ds((j * 16 + i) * _LANES, _LANES)]
            for j, acc in enumerate(accs)
        )

    g = lax.fori_loop(
        0, 16, mbody,
        tuple(jnp.zeros((_LANES,), jnp.int32) for _ in range(16)),
    )

    gt = [jnp.sum(gj) for gj in g]

    zero = jnp.int32(0)
    s_after = zero
    sel_j = zero
    sel_after = zero
    for j in reversed(range(16)):
        crossing = jnp.logical_and(s_after < kk, s_after + gt[j] >= kk)
        sel_j = jnp.where(crossing, jnp.int32(j), sel_j)
        sel_after = jnp.where(crossing, s_after, sel_after)
        s_after = s_after + gt[j]

    base = sel_j * (16 * _LANES)
    tb = [jnp.sum(hist_v[pl.ds(base + i * _LANES, _LANES)]) for i in range(16)]
    s_after2 = sel_after
    sel_i = zero
    count_above = zero
    h_at = zero
    for i in reversed(range(16)):
        crossing = jnp.logical_and(s_after2 < kk, s_after2 + tb[i] >= kk)
        sel_i = jnp.where(crossing, jnp.int32(i), sel_i)
        count_above = jnp.where(crossing, s_after2, count_above)
        h_at = jnp.where(crossing, tb[i], h_at)
        s_after2 = s_after2 + tb[i]

    b = sel_j * 16 + sel_i
    return b, count_above, h_at


def _select_and_mask_row(row_v, cand_a, cand_b, hist_v):
    lane_ids = lax.iota(jnp.int32, _LANES)
    ones = jnp.ones((_LANES,), jnp.int32)
    zeros_i = jnp.zeros((_LANES,), jnp.int32)
    hi_mask = jnp.int32(_HI)
    kk = jnp.int32(_K)

    _zero_hist(hist_v)

    @plsc.parallel_loop(0, _CHUNKS, unroll=_UNROLL)
    def _hist1(i):
        xv = row_v[pl.ds(i * _LANES, _LANES)]
        u = lax.bitcast_convert_type(xv, jnp.int32) & hi_mask
        b = u >> 23
        plsc.addupdate_scatter(hist_v, [b * _LANES + lane_ids], ones)

    b1, ca1, m1 = _level_scan(hist_v, kk)
    kk = kk - ca1
    b1_vec = jnp.full((_LANES,), b1, jnp.int32)

    @plsc.parallel_loop(0, _CHUNKS, unroll=_UNROLL, carry=jnp.int32(0))
    def _compact1(i, cnt):
        xv = row_v[pl.ds(i * _LANES, _LANES)]
        u = lax.bitcast_convert_type(xv, jnp.int32) & hi_mask
        eq = (u >> 23) == b1_vec
        eqi = eq.astype(jnp.int32)
        prefix = plsc.cumsum(eqi)
        off = jnp.where(eq, cnt + prefix - 1, _DUMP + lane_ids)
        plsc.store_scatter(cand_a, [off], u)
        pc = plsc.all_reduce_population_count(eq)
        return cnt + jnp.max(pc)

    def _small_level(src, dst, m, kk, shift, bmask, compact):
        n = (m + 15) >> 4
        m_vec = jnp.full((_LANES,), m, jnp.int32)
        _zero_hist(hist_v)

        @plsc.parallel_loop(0, n, unroll=2, carry=jnp.int32(0))
        def _histS(i, c):
            u = src[pl.ds(i * _LANES, _LANES)]
            live = (i * _LANES + lane_ids) < m_vec
            b = (u >> shift) & bmask
            vals = jnp.where(live, ones, zeros_i)
            plsc.addupdate_scatter(hist_v, [b * _LANES + lane_ids], vals)
            return c

        b, ca, h_at = _level_scan(hist_v, kk)
        if compact:
            b_vec = jnp.full((_LANES,), b, jnp.int32)

            @plsc.parallel_loop(0, n, unroll=2, carry=jnp.int32(0))
            def _compactS(i, cnt):
                u = src[pl.ds(i * _LANES, _LANES)]
                live = (i * _LANES + lane_ids) < m_vec
                eq = jnp.logical_and(((u >> shift) & bmask) == b_vec, live)
                eqi = eq.astype(jnp.int32)
                prefix = plsc.cumsum(eqi)
                off = jnp.where(eq, cnt + prefix - 1, _DUMP + lane_ids)
                plsc.store_scatter(dst, [off], u)
                pc = plsc.all_reduce_population_count(eq)
                return cnt + jnp.max(pc)

        return b, kk - ca, h_at

    bm8 = jnp.int32(0xFF)
    b2, kk, m2 = _small_level(cand_a, cand_b, m1, kk, 15, bm8, True)
    b3, kk, m3 = _small_level(cand_b, cand_a, m2, kk, 7, bm8, True)
    b4, kk4, h_at = _small_level(cand_a, cand_b, m3, kk, 0, jnp.int32(0x7F),
                                 False)

    t = (((((b1 << 8) | b2) << 8) | b3) << 7) | b4
    e = h_at - kk4

    tv = jnp.full((_LANES,), t, jnp.int32)
    zf = jnp.zeros((_LANES,), jnp.float32)

    @plsc.parallel_loop(0, _CHUNKS, unroll=_UNROLL)
    def _out(i):
        sl = pl.ds(i * _LANES, _LANES)
        xv = row_v[sl]
        u = lax.bitcast_convert_type(xv, jnp.int32) & hi_mask
        row_v[sl] = jnp.where(u >= tv, xv, zf)

    @pl.when(e > 0)
    def _fixup():
        e_vec = jnp.full((_LANES,), e, jnp.int32)

        def fbody(i, cnt):
            sl = pl.ds(i * _LANES, _LANES)
            xv = row_v[sl]
            u = lax.bitcast_convert_type(xv, jnp.int32) & hi_mask
            eq = u == tv
            inc = plsc.cumsum(eq.astype(jnp.int32))
            occ = cnt + inc
            row_v[sl] = jnp.where(
                jnp.logical_and(eq, occ <= e_vec), zf, xv
            )
            return cnt + jnp.max(inc)

        lax.fori_loop(0, _CHUNKS, fbody, jnp.int32(0))


def _sc_body(x_hbm, out_hbm, row_v, cand_a, cand_b, hist_v):
    wid = lax.axis_index("c") * _NS + lax.axis_index("s")

    def rbody(rr, c):
        row = wid * _ROWS_PER_W + rr
        pltpu.sync_copy(x_hbm.at[row], row_v)
        _select_and_mask_row(row_v, cand_a, cand_b, hist_v)
        pltpu.sync_copy(row_v, out_hbm.at[row])
        return c

    lax.fori_loop(0, _ROWS_PER_W, rbody, 0)


def kernel(x):
    mesh = plsc.VectorSubcoreMesh(core_axis_name="c", subcore_axis_name="s")
    f = pl.kernel(
        _sc_body,
        out_type=jax.ShapeDtypeStruct((_B, _N), jnp.float32),
        mesh=mesh,
        compiler_params=pltpu.CompilerParams(needs_layout_passes=False),
        scratch_types=[
            pltpu.VMEM((_N,), jnp.float32),
            pltpu.VMEM((_N + _LANES,), jnp.int32),
            pltpu.VMEM((_N + _LANES,), jnp.int32),
            pltpu.VMEM((_LANES * 256,), jnp.int32),
        ],
    )
    return f(x)
